# baseline (speedup 1.0000x reference)
import functools

import jax
import jax.numpy as jnp
import numpy as np
from jax import lax
from jax.experimental import pallas as pl
from jax.experimental.pallas import tpu as pltpu
from jax.experimental.pallas import tpu_sc as plsc

EPS = float(np.finfo(np.float32).eps)
NS = 2000
NSP = 2048
Y0 = 0.9999
STEP = 2.0 * Y0 / (NS - 1)
RSQRT2PI = float(1.0 / np.sqrt(2.0 * np.pi))
R = 16384
NW = 32
RPT = R // NW
NSUP = 8
NCH = 16
L = 16
RPAD = RPT + L


def _tables_1d():
    i = np.minimum(np.arange(NSP), NS - 1).astype(np.float64)
    x = (i * STEP - Y0).astype(np.float32)
    t = 0.5 * np.log((1.0 + x) / (1.0 - x) + EPS, dtype=np.float32)
    coef = (RSQRT2PI / (1.0 - x * x)).astype(np.float32)
    lcoef = np.log(coef).astype(np.float32)
    lcoef[NS:] = -1.0e4
    return jnp.asarray(t), jnp.asarray(lcoef)


def _lanesum(v):
    t0 = (((v[0] + v[1]) + (v[2] + v[3])) + ((v[4] + v[5]) + (v[6] + v[7])))
    t1 = (((v[8] + v[9]) + (v[10] + v[11]))
          + ((v[12] + v[13]) + (v[14] + v[15])))
    return t0 + t1


_mesh = plsc.VectorSubcoreMesh(core_axis_name="c", subcore_axis_name="s")


@functools.partial(
    pl.kernel, mesh=_mesh,
    out_type=[jax.ShapeDtypeStruct((R,), jnp.float32),
              jax.ShapeDtypeStruct((R,), jnp.float32)],
    scratch_types=[pltpu.VMEM((NSP,), jnp.float32),
                   pltpu.VMEM((NSP,), jnp.float32),
                   pltpu.VMEM((RPAD,), jnp.float32),
                   pltpu.VMEM((RPAD,), jnp.float32),
                   pltpu.VMEM((RPAD,), jnp.float32),
                   pltpu.VMEM((NSP,), jnp.float32),
                   pltpu.VMEM((RPAD,), jnp.float32),
                   pltpu.VMEM((RPAD,), jnp.float32)],
)
def _sc_sample(t_hbm, lc_hbm, mean_hbm, std_hbm, u_hbm, val_hbm, prob_hbm,
               t_v, lc_v, mean_v, std_v, u_v, pbuf, ov_v, op_v):
    f32 = jnp.float32
    i32 = jnp.int32
    wid = lax.axis_index("s") * 2 + lax.axis_index("c")
    base = wid * RPT
    pltpu.sync_copy(t_hbm, t_v)
    pltpu.sync_copy(lc_hbm, lc_v)
    pltpu.sync_copy(mean_hbm.at[pl.ds(base, RPAD)], mean_v)
    pltpu.sync_copy(std_hbm.at[pl.ds(base, RPAD)], std_v)
    pltpu.sync_copy(u_hbm.at[pl.ds(base, RPAD)], u_v)
    lanes = lax.iota(jnp.int32, L)
    lane0 = lanes == 0

    def row(i, _):
        mean_s = mean_v[pl.ds(i, L)][0]
        stdv = std_v[pl.ds(i, L)] + EPS
        std_s = stdv[0]
        u_s = u_v[pl.ds(i, L)][0]
        r = (1.0 / stdv)[0]
        an = -0.5 * r * r

        tots = []
        for g1 in range(NSUP):
            def chunk(c, acc, g1=g1):
                o = g1 * (NCH * L) + c * L
                z = t_v[pl.ds(o, L)] - mean_s
                p = jnp.exp(z * z * an + lc_v[pl.ds(o, L)])
                pbuf[pl.ds(o, L)] = p
                return acc + p
            accv = lax.fori_loop(0, NCH, chunk, jnp.zeros((L,), f32))
            tots.append(_lanesum(accv))

        sp = tots[0]
        for g1 in range(1, NSUP):
            sp = sp + tots[g1]
        s = r * sp
        upp = u_s * (s + EPS) * std_s

        pfx = jnp.zeros((), f32)
        g1s = jnp.zeros((), i32)
        off1 = jnp.zeros((), f32)
        for g1 in range(NSUP):
            pfx = pfx + tots[g1]
            m = pfx <= upp
            g1s = g1s + m.astype(i32)
            off1 = jnp.where(m, pfx, off1)
        found = g1s < NSUP
        g1c = jnp.minimum(g1s, NSUP - 1)

        def walk2(c, st):
            pfx2, gsel, off = st
            pv = pbuf[pl.ds(g1c * (NCH * L) + c * L, L)]
            p2 = pfx2 + _lanesum(pv)
            m2 = p2 <= upp
            return (p2, gsel + m2.astype(i32), jnp.where(m2, p2, off))
        _, g2s, off2 = lax.fori_loop(0, NCH, walk2,
                                     (off1, jnp.zeros((), i32), off1))
        g2c = jnp.minimum(g2s, NCH - 1)

        pv = pbuf[pl.ds(g1c * (NCH * L) + g2c * L, L)]
        pfx3 = off2
        cnt = jnp.zeros((), i32)
        psel = jnp.zeros((), f32)
        done = jnp.zeros((), jnp.bool_)
        for c in range(L):
            pe = pv[c]
            pfx3 = pfx3 + pe
            m3 = pfx3 <= upp
            cnt = cnt + m3.astype(i32)
            take = jnp.logical_and(jnp.logical_not(m3),
                                   jnp.logical_not(done))
            psel = jnp.where(take, pe, psel)
            done = jnp.logical_or(done, jnp.logical_not(m3))
        idxr = g1c * (NCH * L) + g2c * L + cnt
        idx = jnp.where(found, jnp.minimum(idxr, NS - 1), 0)
        val = idx.astype(f32) * STEP - Y0
        p0 = pbuf[pl.ds(0, L)][0]
        pnum = jnp.where(found, psel, p0) * r

        wv = ov_v[pl.ds(i, L)]
        ov_v[pl.ds(i, L)] = jnp.where(lane0, val, wv)
        wp = op_v[pl.ds(i, L)]
        op_v[pl.ds(i, L)] = (jnp.where(lane0, pnum, wp)
                             / jnp.where(lane0, s + EPS, 1.0))
        return 0

    lax.fori_loop(0, RPT, row, 0)
    pltpu.sync_copy(ov_v.at[pl.ds(0, RPT)], val_hbm.at[pl.ds(base, RPT)])
    pltpu.sync_copy(op_v.at[pl.ds(0, RPT)], prob_hbm.at[pl.ds(base, RPT)])


@functools.partial(jax.jit, static_argnames=())
def kernel(mean, std, uniform_samples):
    b, a = mean.shape
    t_tab, lc_tab = _tables_1d()
    m = jnp.pad(mean.reshape(R), (0, L))
    s = jnp.pad(std.reshape(R), (0, L))
    u = jnp.pad(uniform_samples.reshape(R), (0, L))
    vals, probs = _sc_sample(t_tab, lc_tab, m, s, u)
    return vals.reshape(b, a), probs.reshape(b, a)


# speedup vs baseline: 5.4711x; 5.4711x over previous
import functools

import jax
import jax.numpy as jnp
import numpy as np
from jax import lax
from jax.experimental import pallas as pl
from jax.experimental.pallas import tpu as pltpu
from jax.experimental.pallas import tpu_sc as plsc
from jax.experimental.pallas import tpu as pltpu

EPS = float(np.finfo(np.float32).eps)
NS = 2000
NSP = 2048
NCHUNK = 16
NL = 128
Y0 = 0.9999
STEP = 2.0 * Y0 / (NS - 1)
RSQRT2PI = float(1.0 / np.sqrt(2.0 * np.pi))


def _tables():
    i = np.minimum(np.arange(NSP), NS - 1).astype(np.float64)
    x = (i * STEP - Y0).astype(np.float32)
    t = 0.5 * np.log((1.0 + x) / (1.0 - x) + EPS, dtype=np.float32)
    coef = (RSQRT2PI / (1.0 - x * x)).astype(np.float32)
    lcoef = np.log(coef).astype(np.float32)
    lcoef[NS:] = -np.inf
    return (jnp.asarray(t.astype(np.float32)).reshape(NSP, 1),
            jnp.asarray(lcoef).reshape(NSP, 1))


def _body(t_ref, c_ref, mean_ref, std_ref, u_ref, val_ref, prob_ref):
    f32 = jnp.float32
    i32 = jnp.int32

    t = t_ref[...]
    lcoef = c_ref[...]
    mean = mean_ref[0]
    std = std_ref[0] + EPS
    u = u_ref[0]
    r = 1.0 / std
    a = -0.5 * r * r

    z = t - mean
    raw = jnp.exp(z * z * a + lcoef) * r

    cs = [jnp.sum(raw[g * NL:(g + 1) * NL, :], axis=0, keepdims=True)
          for g in range(NCHUNK)]
    s = cs[0]
    for g in range(1, NCHUNK):
        s = s + cs[g]
    up = u * (s + EPS)

    acc = jnp.zeros_like(s)
    gst = jnp.zeros(s.shape, i32)
    off = jnp.zeros_like(s)
    for g in range(NCHUNK):
        acc = acc + cs[g]
        m = acc <= up
        gst = gst + m.astype(i32)
        off = off + jnp.where(m, cs[g], 0.0)
    found = gst < NCHUNK
    gs = jnp.minimum(gst, NCHUNK - 1)

    sel = jnp.where(gs == 0, raw[0:NL, :], 0.0)
    for g in range(1, NCHUNK):
        sel = sel + jnp.where(gs == g, raw[g * NL:(g + 1) * NL, :], 0.0)

    ltri = (jax.lax.broadcasted_iota(i32, (NL, NL), 0)
            >= jax.lax.broadcasted_iota(i32, (NL, NL), 1)).astype(f32)
    within = jax.lax.dot_general(ltri, sel, (((1,), (0,)), ((), ())),
                                 preferred_element_type=f32,
                                 precision=jax.lax.Precision.HIGHEST)
    cdfsel = within + off
    cnt = jnp.sum((cdfsel <= up).astype(i32), axis=0, keepdims=True)

    idx = jnp.where(found, gst * NL + cnt, 0)
    idx = jnp.minimum(idx, NS - 1)
    val_ref[0] = idx.astype(f32) * STEP - Y0

    sub = jax.lax.broadcasted_iota(i32, (NL, 1), 0)
    praw = jnp.sum(jnp.where(sub == cnt, sel, 0.0), axis=0, keepdims=True)
    p0 = raw[0:1, :]
    prob_ref[0] = jnp.where(found, praw, p0) / (s + EPS)


def _tc_part(m, s, u, rows):
    rb = 1024
    nb = rows // rb
    m = m.reshape(nb, 1, rb)
    s = s.reshape(nb, 1, rb)
    u = u.reshape(nb, 1, rb)
    t_tab, c_tab = _tables()
    tab = pl.BlockSpec((NSP, 1), lambda i: (0, 0))
    col = pl.BlockSpec((1, 1, rb), lambda i: (i, 0, 0))
    vals, probs = pl.pallas_call(
        _body,
        grid=(nb,),
        in_specs=[tab, tab, col, col, col],
        out_specs=[col, col],
        out_shape=[
            jax.ShapeDtypeStruct((nb, 1, rb), jnp.float32),
            jax.ShapeDtypeStruct((nb, 1, rb), jnp.float32),
        ],
    )(t_tab, c_tab, m, s, u)
    return vals.reshape(rows), probs.reshape(rows)


R_SC = 2048
NW = 32
RPT = R_SC // NW
NSUP = 8
NCH = 16
L = 16
RPAD = RPT + L


def _tables_1d():
    i = np.minimum(np.arange(NSP), NS - 1).astype(np.float64)
    x = (i * STEP - Y0).astype(np.float32)
    t = 0.5 * np.log((1.0 + x) / (1.0 - x) + EPS, dtype=np.float32)
    coef = (RSQRT2PI / (1.0 - x * x)).astype(np.float32)
    lcoef = np.log(coef).astype(np.float32)
    lcoef[NS:] = -1.0e4
    return jnp.asarray(t), jnp.asarray(lcoef)


def _lanesum(v):
    t0 = (((v[0] + v[1]) + (v[2] + v[3])) + ((v[4] + v[5]) + (v[6] + v[7])))
    t1 = (((v[8] + v[9]) + (v[10] + v[11]))
          + ((v[12] + v[13]) + (v[14] + v[15])))
    return t0 + t1


_mesh = plsc.VectorSubcoreMesh(core_axis_name="c", subcore_axis_name="s")


@functools.partial(
    pl.kernel, mesh=_mesh,
    out_type=[jax.ShapeDtypeStruct((R_SC,), jnp.float32),
              jax.ShapeDtypeStruct((R_SC,), jnp.float32)],
    scratch_types=[pltpu.VMEM((NSP,), jnp.float32),
                   pltpu.VMEM((NSP,), jnp.float32),
                   pltpu.VMEM((RPAD,), jnp.float32),
                   pltpu.VMEM((RPAD,), jnp.float32),
                   pltpu.VMEM((RPAD,), jnp.float32),
                   pltpu.VMEM((NSP,), jnp.float32),
                   pltpu.VMEM((RPAD,), jnp.float32),
                   pltpu.VMEM((RPAD,), jnp.float32)],
)
def _sc_sample(t_hbm, lc_hbm, mean_hbm, std_hbm, u_hbm, val_hbm, prob_hbm,
               t_v, lc_v, mean_v, std_v, u_v, pbuf, ov_v, op_v):
    f32 = jnp.float32
    i32 = jnp.int32
    wid = lax.axis_index("s") * 2 + lax.axis_index("c")
    base = wid * RPT
    pltpu.sync_copy(t_hbm, t_v)
    pltpu.sync_copy(lc_hbm, lc_v)
    pltpu.sync_copy(mean_hbm.at[pl.ds(base, RPAD)], mean_v)
    pltpu.sync_copy(std_hbm.at[pl.ds(base, RPAD)], std_v)
    pltpu.sync_copy(u_hbm.at[pl.ds(base, RPAD)], u_v)
    lanes = lax.iota(jnp.int32, L)
    lane0 = lanes == 0

    def row(i, _):
        mean_s = mean_v[pl.ds(i, L)][0]
        stdv = std_v[pl.ds(i, L)] + EPS
        std_s = stdv[0]
        u_s = u_v[pl.ds(i, L)][0]
        r = (1.0 / stdv)[0]
        an = -0.5 * r * r

        tots = []
        for g1 in range(NSUP):
            def chunk(c, acc, g1=g1):
                o = g1 * (NCH * L) + c * L
                z = t_v[pl.ds(o, L)] - mean_s
                p = jnp.exp(z * z * an + lc_v[pl.ds(o, L)])
                pbuf[pl.ds(o, L)] = p
                return acc + p
            accv = lax.fori_loop(0, NCH, chunk, jnp.zeros((L,), f32))
            tots.append(_lanesum(accv))

        sp = tots[0]
        for g1 in range(1, NSUP):
            sp = sp + tots[g1]
        s = r * sp
        upp = u_s * (s + EPS) * std_s

        pfx = jnp.zeros((), f32)
        g1s = jnp.zeros((), i32)
        off1 = jnp.zeros((), f32)
        for g1 in range(NSUP):
            pfx = pfx + tots[g1]
            m = pfx <= upp
            g1s = g1s + m.astype(i32)
            off1 = jnp.where(m, pfx, off1)
        found = g1s < NSUP
        g1c = jnp.minimum(g1s, NSUP - 1)

        def walk2(c, st):
            pfx2, gsel, off = st
            pv = pbuf[pl.ds(g1c * (NCH * L) + c * L, L)]
            p2 = pfx2 + _lanesum(pv)
            m2 = p2 <= upp
            return (p2, gsel + m2.astype(i32), jnp.where(m2, p2, off))
        _, g2s, off2 = lax.fori_loop(0, NCH, walk2,
                                     (off1, jnp.zeros((), i32), off1))
        g2c = jnp.minimum(g2s, NCH - 1)

        pv = pbuf[pl.ds(g1c * (NCH * L) + g2c * L, L)]
        pfx3 = off2
        cnt = jnp.zeros((), i32)
        psel = jnp.zeros((), f32)
        done = jnp.zeros((), jnp.bool_)
        for c in range(L):
            pe = pv[c]
            pfx3 = pfx3 + pe
            m3 = pfx3 <= upp
            cnt = cnt + m3.astype(i32)
            take = jnp.logical_and(jnp.logical_not(m3),
                                   jnp.logical_not(done))
            psel = jnp.where(take, pe, psel)
            done = jnp.logical_or(done, jnp.logical_not(m3))
        idxr = g1c * (NCH * L) + g2c * L + cnt
        idx = jnp.where(found, jnp.minimum(idxr, NS - 1), 0)
        val = idx.astype(f32) * STEP - Y0
        p0 = pbuf[pl.ds(0, L)][0]
        pnum = jnp.where(found, psel, p0) * r

        wv = ov_v[pl.ds(i, L)]
        ov_v[pl.ds(i, L)] = jnp.where(lane0, val, wv)
        wp = op_v[pl.ds(i, L)]
        op_v[pl.ds(i, L)] = (jnp.where(lane0, pnum, wp)
                             / jnp.where(lane0, s + EPS, 1.0))
        return 0

    lax.fori_loop(0, RPT, row, 0)
    pltpu.sync_copy(ov_v.at[pl.ds(0, RPT)], val_hbm.at[pl.ds(base, RPT)])
    pltpu.sync_copy(op_v.at[pl.ds(0, RPT)], prob_hbm.at[pl.ds(base, RPT)])


def _sc_part(m, s, u):
    t_tab, lc_tab = _tables_1d()
    vals, probs = _sc_sample(t_tab, lc_tab, jnp.pad(m, (0, L)),
                             jnp.pad(s, (0, L)), jnp.pad(u, (0, L)))
    return vals, probs


TC_ROWS = 16384 - R_SC


@functools.partial(jax.jit, static_argnames=())
def kernel(mean, std, uniform_samples):
    b, a = mean.shape
    rows = b * a
    m = mean.reshape(rows)
    s = std.reshape(rows)
    u = uniform_samples.reshape(rows)
    tv, tp = _tc_part(m[:TC_ROWS], s[:TC_ROWS], u[:TC_ROWS], TC_ROWS)
    sv, sp_ = _sc_part(m[TC_ROWS:], s[TC_ROWS:], u[TC_ROWS:])
    vals = jnp.concatenate([tv, sv]).reshape(b, a)
    probs = jnp.concatenate([tp, sp_]).reshape(b, a)
    return vals, probs


# speedup vs baseline: 6.8902x; 1.2594x over previous
import functools

import jax
import jax.numpy as jnp
import numpy as np
from jax.experimental import pallas as pl
from jax.experimental.pallas import tpu as pltpu

EPS = float(np.finfo(np.float32).eps)
NS = 2000
NSP = 2048
NCHUNK = 16
NL = 128
Y0 = 0.9999
STEP = 2.0 * Y0 / (NS - 1)
RSQRT2PI = float(1.0 / np.sqrt(2.0 * np.pi))


def _tables():
    i = np.minimum(np.arange(NSP), NS - 1).astype(np.float64)
    x = (i * STEP - Y0).astype(np.float32)
    t = 0.5 * np.log((1.0 + x) / (1.0 - x) + EPS, dtype=np.float32)
    coef = (RSQRT2PI / (1.0 - x * x)).astype(np.float32)
    lcoef = np.log(coef).astype(np.float32)
    lcoef[NS:] = -np.inf
    return (jnp.asarray(t.astype(np.float32)).reshape(NSP, 1),
            jnp.asarray(lcoef).reshape(NSP, 1))


def _body(t_ref, c_ref, mean_ref, std_ref, u_ref, val_ref, prob_ref):
    f32 = jnp.float32
    i32 = jnp.int32

    t = t_ref[...]
    lcoef = c_ref[...]
    mean = mean_ref[0]
    std = std_ref[0] + EPS
    u = u_ref[0]
    r = 1.0 / std
    a = -0.5 * r * r

    z = t - mean
    raw = jnp.exp(z * z * a + lcoef) * r

    cs = [jnp.sum(raw[g * NL:(g + 1) * NL, :], axis=0, keepdims=True)
          for g in range(NCHUNK)]
    s = cs[0]
    for g in range(1, NCHUNK):
        s = s + cs[g]
    up = u * (s + EPS)

    acc = jnp.zeros_like(s)
    gst = jnp.zeros(s.shape, i32)
    off = jnp.zeros_like(s)
    for g in range(NCHUNK):
        acc = acc + cs[g]
        m = acc <= up
        gst = gst + m.astype(i32)
        off = off + jnp.where(m, cs[g], 0.0)
    found = gst < NCHUNK
    gs = jnp.minimum(gst, NCHUNK - 1)

    sel = jnp.where(gs == 0, raw[0:NL, :], 0.0)
    for g in range(1, NCHUNK):
        sel = sel + jnp.where(gs == g, raw[g * NL:(g + 1) * NL, :], 0.0)

    ltri = (jax.lax.broadcasted_iota(i32, (NL, NL), 0)
            >= jax.lax.broadcasted_iota(i32, (NL, NL), 1)).astype(f32)
    within = jax.lax.dot_general(ltri, sel, (((1,), (0,)), ((), ())),
                                 preferred_element_type=f32,
                                 precision=jax.lax.Precision.HIGHEST)
    cdfsel = within + off
    cnt = jnp.sum((cdfsel <= up).astype(i32), axis=0, keepdims=True)

    idx = jnp.where(found, gst * NL + cnt, 0)
    idx = jnp.minimum(idx, NS - 1)
    val_ref[0] = idx.astype(f32) * STEP - Y0

    sub = jax.lax.broadcasted_iota(i32, (NL, 1), 0)
    praw = jnp.sum(jnp.where(sub == cnt, sel, 0.0), axis=0, keepdims=True)
    p0 = raw[0:1, :]
    prob_ref[0] = jnp.where(found, praw, p0) / (s + EPS)


@functools.partial(jax.jit, static_argnames=())
def kernel(mean, std, uniform_samples):
    b, a = mean.shape
    rows = b * a
    rb = 1024
    nb = rows // rb
    m = mean.reshape(nb, 1, rb)
    s = std.reshape(nb, 1, rb)
    u = uniform_samples.reshape(nb, 1, rb)
    t_tab, c_tab = _tables()
    tab = pl.BlockSpec((NSP, 1), lambda i: (0, 0))
    col = pl.BlockSpec((1, 1, rb), lambda i: (i, 0, 0))
    vals, probs = pl.pallas_call(
        _body,
        grid=(nb,),
        in_specs=[tab, tab, col, col, col],
        out_specs=[col, col],
        out_shape=[
            jax.ShapeDtypeStruct((nb, 1, rb), jnp.float32),
            jax.ShapeDtypeStruct((nb, 1, rb), jnp.float32),
        ],
    )(t_tab, c_tab, m, s, u)
    return vals.reshape(b, a), probs.reshape(b, a)
